# baseline (speedup 1.0000x reference)
import functools
import math

import jax
import jax.numpy as jnp
from jax import lax
from jax.experimental import pallas as pl
from jax.experimental.pallas import tpu as pltpu
from jax.experimental.pallas import tpu_sc as plsc

N = 10000
FEAT = 128
K = int(math.ceil(0.5 * N))
E = 320000

NC = 2
NS = 16
NW = NC * NS
CH = 80
RPW = 128
EPW = CH * RPW
EP = EPW * NW
EROWS = EP // CH
NA = 10240
NTRASH = NA - N
RPN = NA // NS

_mesh = plsc.VectorSubcoreMesh(core_axis_name="c", subcore_axis_name="s")


def _worker_id():
    return lax.axis_index("c") * NS + lax.axis_index("s")


_NLP = pltpu.CompilerParams(needs_layout_passes=False)


@functools.partial(
    pl.kernel,
    mesh=_mesh,
    out_type=jax.ShapeDtypeStruct((NC, NA, FEAT), jnp.float32),
    scratch_types=[
        pltpu.VMEM((RPW // 2, CH), jnp.int32),
        pltpu.VMEM((RPW // 2, CH), jnp.int32),
        pltpu.VMEM((CH, FEAT), jnp.float32),
        pltpu.VMEM((CH, FEAT), jnp.float32),
        pltpu.VMEM((16,), jnp.int32),
        pltpu.VMEM_SHARED((NA, FEAT), jnp.float32),
        pltpu.SemaphoreType.DMA,
        pltpu.SemaphoreType.DMA,
        pltpu.SemaphoreType.DMA,
        pltpu.SemaphoreType.DMA,
    ],
    compiler_params=_NLP,
)
def _sc_scatter(h_hbm, ns_hbm, nd_hbm, cnt_hbm, z_hbm, out_hbm,
                ns_slab, nd_slab, bufa, bufb, cv, acc, gsa, gsb, ssa, ssb):
    cid = lax.axis_index("c")
    sid = lax.axis_index("s")
    w = cid * NS + sid

    pltpu.sync_copy(z_hbm, acc.at[pl.ds(sid * RPN, RPN), :])
    plsc.subcore_barrier()

    pltpu.sync_copy(cnt_hbm.at[w], cv)
    nch = jnp.max(cv[...])
    hh = RPW // 2

    def _gcopy(j, buf, sem):
        return pltpu.make_async_copy(h_hbm.at[ns_slab.at[j]], buf, sem)

    def _swait(buf, sem):
        pltpu.make_async_copy(buf, acc.at[nd_slab.at[0]], sem).wait()

    for half in range(2):
        j0 = half * hh

        @pl.when(j0 < nch)
        def _():
            pltpu.sync_copy(ns_hbm.at[pl.ds(w * RPW + j0, hh), :], ns_slab)
            pltpu.sync_copy(nd_hbm.at[pl.ds(w * RPW + j0, hh), :], nd_slab)
            _gcopy(0, bufa, gsa).start()

            def _stage(j, buf, gsem, ssem, nbuf, ngsem, nssem):
                @pl.when(j0 + j < nch)
                def _():
                    _gcopy(j, buf, gsem).wait()
                    pltpu.async_copy(buf, acc.at[nd_slab.at[j]], ssem, add=True)

                    @pl.when((j0 + j + 1 < nch) & (j + 1 < hh))
                    def _():
                        @pl.when(j > 0)
                        def _():
                            _swait(nbuf, nssem)
                        _gcopy(j + 1, nbuf, ngsem).start()

            def _pair(i, carry):
                j = i * 2
                _stage(j, bufa, gsa, ssa, bufb, gsb, ssb)
                _stage(j + 1, bufb, gsb, ssb, bufa, gsa, ssa)
                return carry
            lax.fori_loop(0, hh // 2, _pair, 0)

            nloc = jnp.minimum(nch - j0, hh)

            @pl.when(nloc >= 2)
            def _():
                _swait(bufa, ssa)
                _swait(bufb, ssb)

            @pl.when(nloc == 1)
            def _():
                _swait(bufa, ssa)
    plsc.subcore_barrier()

    for t in range(RPN // CH):
        r0 = sid * RPN + t * CH
        pltpu.sync_copy(acc.at[pl.ds(r0, CH), :], out_hbm.at[cid, pl.ds(r0, CH), :])


@functools.partial(
    pl.kernel,
    mesh=_mesh,
    out_type=jax.ShapeDtypeStruct((NC, NA), jnp.float32),
    scratch_types=[
        pltpu.VMEM((RPW, CH), jnp.int32),
        pltpu.VMEM((CH,), jnp.float32),
        pltpu.VMEM((16,), jnp.int32),
        pltpu.VMEM_SHARED((NA,), jnp.float32),
    ],
    compiler_params=_NLP,
)
def _sc_degree(nd_hbm, cnt_hbm, z_hbm, out_hbm, nd_slab, obuf, cv, acc):
    cid = lax.axis_index("c")
    sid = lax.axis_index("s")
    w = cid * NS + sid

    o16 = jnp.ones((16,), jnp.float32)
    for c in range(CH // 16):
        obuf[pl.ds(c * 16, 16)] = o16
    pltpu.sync_copy(z_hbm, acc.at[pl.ds(sid * RPN, RPN)])
    plsc.subcore_barrier()

    pltpu.sync_copy(nd_hbm.at[pl.ds(w * RPW, RPW), :], nd_slab)
    pltpu.sync_copy(cnt_hbm.at[w], cv)
    nch = jnp.max(cv[...])

    def _row(j, carry):
        @pl.when(j < nch)
        def _():
            pltpu.sync_copy(obuf, acc.at[nd_slab.at[j]], add=True)
        return carry
    lax.fori_loop(0, RPW, _row, 0)
    plsc.subcore_barrier()

    pltpu.sync_copy(acc.at[pl.ds(sid * RPN, RPN)],
                    out_hbm.at[cid, pl.ds(sid * RPN, RPN)])


@functools.partial(
    pl.kernel,
    mesh=_mesh,
    out_type=(
        jax.ShapeDtypeStruct((EP,), jnp.int32),
        jax.ShapeDtypeStruct((EP,), jnp.int32),
        jax.ShapeDtypeStruct((NW, 16), jnp.int32),
    ),
    scratch_types=[
        pltpu.VMEM((NA,), jnp.int32),
        pltpu.VMEM((EPW,), jnp.int32),
        pltpu.VMEM((EPW,), jnp.int32),
        pltpu.VMEM((EPW + CH,), jnp.int32),
        pltpu.VMEM((EPW + CH,), jnp.int32),
        pltpu.VMEM((16,), jnp.int32),
    ],
    compiler_params=_NLP,
)
def _sc_compact(mmap_hbm, src_hbm, dst_hbm, ns_out, nd_out, cnt_out,
                mm, sfl, dfl, nsb, ndb, cbuf):
    cid = lax.axis_index("c")
    sid = lax.axis_index("s")
    w = cid * NS + sid
    lane = lax.iota(jnp.int32, 16)

    pltpu.sync_copy(mmap_hbm, mm)
    pltpu.sync_copy(src_hbm.at[pl.ds(w * EPW, EPW)], sfl)
    pltpu.sync_copy(dst_hbm.at[pl.ds(w * EPW, EPW)], dfl)

    def _tile(t, cnt):
        idx = t * 16 + lane
        sv = plsc.load_gather(sfl, [idx])
        dv = plsc.load_gather(dfl, [idx])
        nsv = plsc.load_gather(mm, [sv])
        ndv = plsc.load_gather(mm, [dv])
        val = (nsv >= 0) & (ndv >= 0)
        pos = cnt + jnp.cumsum(jnp.where(val, 1, 0)) - 1
        plsc.store_scatter(nsb, [pos], nsv, mask=val)
        plsc.store_scatter(ndb, [pos], ndv, mask=val)
        return cnt + jnp.max(plsc.all_reduce_population_count(val))
    cnt = lax.fori_loop(0, EPW // 16, _tile, jnp.int32(0))

    for i in range(CH // 16):
        plsc.store_scatter(nsb, [cnt + i * 16 + lane], lane)
        plsc.store_scatter(ndb, [cnt + i * 16 + lane],
                           N + ((w * CH + i * 16 + lane) % NTRASH))

    nchunks = (cnt + (CH - 1)) // CH
    cbuf[...] = jnp.broadcast_to(nchunks, (16,)).astype(jnp.int32)
    pltpu.sync_copy(cbuf, cnt_out.at[w])
    pltpu.sync_copy(nsb.at[pl.ds(0, EPW)], ns_out.at[pl.ds(w * EPW, EPW)])
    pltpu.sync_copy(ndb.at[pl.ds(0, EPW)], nd_out.at[pl.ds(w * EPW, EPW)])


_BR = 1024
_G = NA // _BR


def _node_spec():
    return pl.BlockSpec((_BR, FEAT), lambda i: (i, 0))


def _col_spec():
    return pl.BlockSpec((_BR, 1), lambda i: (i, 0))


def _s_spec():
    return pl.BlockSpec((NC, _BR, FEAT), lambda i: (0, i, 0))


def _w_spec():
    return pl.BlockSpec((FEAT, FEAT), lambda i: (0, 0))


def _b_spec():
    return pl.BlockSpec((1, FEAT), lambda i: (0, 0))


def _tc_first_body(x_ref, w_ref, d_ref, h_ref, dinv_ref):
    deg = 1.0 + d_ref[0] + d_ref[1]
    dinv = 1.0 / jnp.sqrt(deg)
    h = jnp.dot(x_ref[...], w_ref[...], preferred_element_type=jnp.float32)
    h_ref[...] = h * dinv
    dinv_ref[...] = dinv


def _tc_first(x, w1, degp):
    return pl.pallas_call(
        _tc_first_body,
        grid=(_G,),
        in_specs=[_node_spec(), _w_spec(),
                  pl.BlockSpec((NC, _BR, 1), lambda i: (0, i, 0))],
        out_specs=[_node_spec(), _col_spec()],
        out_shape=[jax.ShapeDtypeStruct((NA, FEAT), jnp.float32),
                   jax.ShapeDtypeStruct((NA, 1), jnp.float32)],
    )(x, w1, degp)


def _tc_mid_body(h_ref, s_ref, dinv_ref, w_ref, b_ref, o_ref):
    dinv = dinv_ref[...]
    x = jnp.tanh(dinv * (h_ref[...] + s_ref[0] + s_ref[1]) + b_ref[...])
    o_ref[...] = jnp.dot(x, w_ref[...], preferred_element_type=jnp.float32) * dinv


def _tc_mid(h, s, dinv, w_next, b_prev):
    return pl.pallas_call(
        _tc_mid_body,
        grid=(_G,),
        in_specs=[_node_spec(), _s_spec(), _col_spec(), _w_spec(), _b_spec()],
        out_specs=_node_spec(),
        out_shape=jax.ShapeDtypeStruct((NA, FEAT), jnp.float32),
    )(h, s, dinv, w_next, b_prev)


def _tc_pool_a_body(h_ref, s_ref, dinv_ref, b_ref, pw_ref, x_ref, sc_ref):
    i = pl.program_id(0)
    x = jnp.tanh(dinv_ref[...] * (h_ref[...] + s_ref[0] + s_ref[1]) + b_ref[...])
    x_ref[...] = x
    pw = pw_ref[...]
    nrm = jnp.sqrt(jnp.sum(pw * pw))
    raw = jnp.sum(x * pw, axis=1, keepdims=True) / nrm
    rows = i * _BR + lax.broadcasted_iota(jnp.int32, (_BR, 1), 0)
    sc_ref[...] = jnp.where(rows < N, jnp.tanh(raw), -2.0)


def _tc_pool_a(h, s, dinv, b2, pw_row):
    return pl.pallas_call(
        _tc_pool_a_body,
        grid=(_G,),
        in_specs=[_node_spec(), _s_spec(), _col_spec(), _b_spec(),
                  pl.BlockSpec((1, FEAT), lambda i: (0, 0))],
        out_specs=[_node_spec(), _col_spec()],
        out_shape=[jax.ShapeDtypeStruct((NA, FEAT), jnp.float32),
                   jax.ShapeDtypeStruct((NA, 1), jnp.float32)],
    )(h, s, dinv, b2, pw_row)


def _tc_rank_body(sv_ref, su_ref, mmap_ref, gate_ref):
    i = pl.program_id(0)
    sv = sv_ref[...]
    vidx = i * _BR + lax.broadcasted_iota(jnp.int32, (_BR, 1), 0)
    svb = jnp.broadcast_to(sv, (_BR, 128))
    vidxb = jnp.broadcast_to(vidx, (_BR, 128))
    lanes = lax.broadcasted_iota(jnp.int32, (_BR, 128), 1)

    def _u(r, cnt):
        su = jnp.broadcast_to(su_ref[pl.ds(r, 1), :], (_BR, 128))
        uidx = r * 128 + lanes
        inc = (su > svb) | ((su == svb) & (uidx < vidxb))
        return cnt + jnp.where(inc, 1.0, 0.0)
    cnt = lax.fori_loop(0, NA // 128, _u, jnp.zeros((_BR, 128), jnp.float32))
    rank = jnp.sum(cnt, axis=1, keepdims=True)
    kept = rank < K
    mmap_ref[...] = jnp.where(kept, rank.astype(jnp.int32), -1)
    gate_ref[...] = jnp.where(kept, sv, 0.0)


def _tc_rank(score_col, score_row):
    return pl.pallas_call(
        _tc_rank_body,
        grid=(_G,),
        in_specs=[_col_spec(),
                  pl.BlockSpec((NA // 128, 128), lambda i: (0, 0))],
        out_specs=[_col_spec(), _col_spec()],
        out_shape=[jax.ShapeDtypeStruct((NA, 1), jnp.int32),
                   jax.ShapeDtypeStruct((NA, 1), jnp.float32)],
    )(score_col, score_row)


def _tc_pool_b_body(x_ref, g_ref, d_ref, w_ref, h_ref, dinv_ref):
    deg = 1.0 + d_ref[0] + d_ref[1]
    dinv = 1.0 / jnp.sqrt(deg)
    xo = x_ref[...] * g_ref[...]
    h_ref[...] = jnp.dot(xo, w_ref[...], preferred_element_type=jnp.float32) * dinv
    dinv_ref[...] = dinv


def _tc_pool_b(x2, gate, degp, w3):
    return pl.pallas_call(
        _tc_pool_b_body,
        grid=(_G,),
        in_specs=[_node_spec(), _col_spec(),
                  pl.BlockSpec((NC, _BR, 1), lambda i: (0, i, 0)), _w_spec()],
        out_specs=[_node_spec(), _col_spec()],
        out_shape=[jax.ShapeDtypeStruct((NA, FEAT), jnp.float32),
                   jax.ShapeDtypeStruct((NA, 1), jnp.float32)],
    )(x2, gate, degp, w3)


def _tc_final_body(h_ref, s_ref, dinv_ref, b_ref, o_ref):
    o_ref[...] = dinv_ref[...] * (h_ref[...] + s_ref[0] + s_ref[1]) + b_ref[...]


def _tc_final(h, s, dinv, b5):
    return pl.pallas_call(
        _tc_final_body,
        grid=(_G,),
        in_specs=[_node_spec(), _s_spec(), _col_spec(), _b_spec()],
        out_specs=_node_spec(),
        out_shape=jax.ShapeDtypeStruct((NA, FEAT), jnp.float32),
    )(h, s, dinv, b5)


def kernel(nodes, edges, batch, W1, b1, W2, b2, W3, b3, W4, b4, W5, b5, pw):
    del batch
    f32 = jnp.float32
    src = edges[0]
    dst = edges[1]

    npad = EP - E
    ar = jnp.arange(npad, dtype=jnp.int32)
    srcp = jnp.concatenate([src, ar % N]).reshape(EROWS, CH)
    dstp = jnp.concatenate([dst, N + (ar % NTRASH)]).reshape(EROWS, CH)
    full_cnt = jnp.full((NW, 16), RPW, dtype=jnp.int32)

    nodes_p = jnp.pad(nodes, ((0, NA - N), (0, 0)))
    b1r = b1.reshape(1, FEAT)
    b2r = b2.reshape(1, FEAT)
    b3r = b3.reshape(1, FEAT)
    b4r = b4.reshape(1, FEAT)
    b5r = b5.reshape(1, FEAT)
    pw_row = pw.reshape(1, FEAT)

    zrow = jnp.zeros((RPN, FEAT), dtype=f32)
    zdeg = jnp.zeros((RPN,), dtype=f32)

    deg0p = _sc_degree(dstp, full_cnt, zdeg)
    h1, dinv0 = _tc_first(nodes_p, W1, deg0p.reshape(NC, NA, 1))
    s1 = _sc_scatter(h1, srcp, dstp, full_cnt, zrow)
    h2 = _tc_mid(h1, s1, dinv0, W2, b1r)
    s2 = _sc_scatter(h2, srcp, dstp, full_cnt, zrow)

    x2, score = _tc_pool_a(h2, s2, dinv0, b2r, pw_row)
    mmap, gate = _tc_rank(score, score.reshape(NA // 128, 128))

    nsc, ndc, ccnt = _sc_compact(mmap.reshape(NA), srcp.reshape(EP), dstp.reshape(EP))
    nsc2 = nsc.reshape(EROWS, CH)
    ndc2 = ndc.reshape(EROWS, CH)
    deg1p = _sc_degree(ndc2, ccnt, zdeg)

    h3, dinv1 = _tc_pool_b(x2, gate, deg1p.reshape(NC, NA, 1), W3)
    s3 = _sc_scatter(h3, nsc2, ndc2, ccnt, zrow)
    h4 = _tc_mid(h3, s3, dinv1, W4, b3r)
    s4 = _sc_scatter(h4, nsc2, ndc2, ccnt, zrow)
    h5 = _tc_mid(h4, s4, dinv1, W5, b4r)
    s5 = _sc_scatter(h5, nsc2, ndc2, ccnt, zrow)
    out = _tc_final(h5, s5, dinv1, b5r)
    return out[:N].astype(f32)


# speedup vs baseline: 1.1337x; 1.1337x over previous
import functools
import math

import jax
import jax.numpy as jnp
from jax import lax
from jax.experimental import pallas as pl
from jax.experimental.pallas import tpu as pltpu
from jax.experimental.pallas import tpu_sc as plsc

N = 10000
FEAT = 128
K = int(math.ceil(0.5 * N))
E = 320000

NC = 2
NS = 16
NW = NC * NS
CH = 80
RPW = 128
EPW = CH * RPW
EP = EPW * NW
EROWS = EP // CH
NA = 10240
NTRASH = NA - N
RPN = NA // NS

_mesh = plsc.VectorSubcoreMesh(core_axis_name="c", subcore_axis_name="s")


def _worker_id():
    return lax.axis_index("c") * NS + lax.axis_index("s")


_NLP = pltpu.CompilerParams(needs_layout_passes=False)


@functools.partial(
    pl.kernel,
    mesh=_mesh,
    out_type=jax.ShapeDtypeStruct((NC, NA, FEAT), jnp.float32),
    scratch_types=[
        pltpu.VMEM((RPW // 2, CH), jnp.int32),
        pltpu.VMEM((RPW // 2, CH), jnp.int32),
        pltpu.VMEM((CH, FEAT), jnp.float32),
        pltpu.VMEM((CH, FEAT), jnp.float32),
        pltpu.VMEM((16,), jnp.int32),
        pltpu.VMEM_SHARED((NA, FEAT), jnp.float32),
        pltpu.SemaphoreType.DMA,
        pltpu.SemaphoreType.DMA,
    ],
    compiler_params=_NLP,
)
def _sc_scatter(h_hbm, ns_hbm, nd_hbm, cnt_hbm, z_hbm, out_hbm,
                ns_slab, nd_slab, bufa, bufb, cv, acc, sema, semb):
    cid = lax.axis_index("c")
    sid = lax.axis_index("s")
    w = cid * NS + sid

    pltpu.sync_copy(z_hbm, acc.at[pl.ds(sid * RPN, RPN), :])
    plsc.subcore_barrier()

    pltpu.sync_copy(cnt_hbm.at[w], cv)
    nch = jnp.max(cv[...])
    hh = RPW // 2

    def _gcopy(j, buf, sem):
        return pltpu.make_async_copy(h_hbm.at[ns_slab.at[j]], buf, sem)

    for half in range(2):
        j0 = half * hh

        @pl.when(j0 < nch)
        def _():
            pltpu.sync_copy(ns_hbm.at[pl.ds(w * RPW + j0, hh), :], ns_slab)
            pltpu.sync_copy(nd_hbm.at[pl.ds(w * RPW + j0, hh), :], nd_slab)
            _gcopy(0, bufa, sema).start()

            def _stage(j, buf, sem, nbuf, nsem):
                @pl.when(j0 + j < nch)
                def _():
                    @pl.when(j0 + j + 1 < nch)
                    def _():
                        @pl.when(j + 1 < hh)
                        def _():
                            _gcopy(j + 1, nbuf, nsem).start()
                    _gcopy(j, buf, sem).wait()
                    pltpu.sync_copy(buf, acc.at[nd_slab.at[j]], add=True)

            def _pair(i, carry):
                j = i * 2
                _stage(j, bufa, sema, bufb, semb)
                _stage(j + 1, bufb, semb, bufa, sema)
                return carry
            lax.fori_loop(0, hh // 2, _pair, 0)
    plsc.subcore_barrier()

    for t in range(RPN // CH):
        r0 = sid * RPN + t * CH
        pltpu.sync_copy(acc.at[pl.ds(r0, CH), :], out_hbm.at[cid, pl.ds(r0, CH), :])


@functools.partial(
    pl.kernel,
    mesh=_mesh,
    out_type=jax.ShapeDtypeStruct((NC, NA), jnp.float32),
    scratch_types=[
        pltpu.VMEM((RPW, CH), jnp.int32),
        pltpu.VMEM((CH,), jnp.float32),
        pltpu.VMEM((16,), jnp.int32),
        pltpu.VMEM_SHARED((NA,), jnp.float32),
    ],
    compiler_params=_NLP,
)
def _sc_degree(nd_hbm, cnt_hbm, z_hbm, out_hbm, nd_slab, obuf, cv, acc):
    cid = lax.axis_index("c")
    sid = lax.axis_index("s")
    w = cid * NS + sid

    o16 = jnp.ones((16,), jnp.float32)
    for c in range(CH // 16):
        obuf[pl.ds(c * 16, 16)] = o16
    pltpu.sync_copy(z_hbm, acc.at[pl.ds(sid * RPN, RPN)])
    plsc.subcore_barrier()

    pltpu.sync_copy(nd_hbm.at[pl.ds(w * RPW, RPW), :], nd_slab)
    pltpu.sync_copy(cnt_hbm.at[w], cv)
    nch = jnp.max(cv[...])

    def _row(j, carry):
        @pl.when(j < nch)
        def _():
            pltpu.sync_copy(obuf, acc.at[nd_slab.at[j]], add=True)
        return carry
    lax.fori_loop(0, RPW, _row, 0)
    plsc.subcore_barrier()

    pltpu.sync_copy(acc.at[pl.ds(sid * RPN, RPN)],
                    out_hbm.at[cid, pl.ds(sid * RPN, RPN)])


@functools.partial(
    pl.kernel,
    mesh=_mesh,
    out_type=(
        jax.ShapeDtypeStruct((EP,), jnp.int32),
        jax.ShapeDtypeStruct((EP,), jnp.int32),
        jax.ShapeDtypeStruct((NW, 16), jnp.int32),
    ),
    scratch_types=[
        pltpu.VMEM((NA,), jnp.int32),
        pltpu.VMEM((EPW,), jnp.int32),
        pltpu.VMEM((EPW,), jnp.int32),
        pltpu.VMEM((EPW + CH,), jnp.int32),
        pltpu.VMEM((EPW + CH,), jnp.int32),
        pltpu.VMEM((16,), jnp.int32),
    ],
    compiler_params=_NLP,
)
def _sc_compact(mmap_hbm, src_hbm, dst_hbm, ns_out, nd_out, cnt_out,
                mm, sfl, dfl, nsb, ndb, cbuf):
    cid = lax.axis_index("c")
    sid = lax.axis_index("s")
    w = cid * NS + sid
    lane = lax.iota(jnp.int32, 16)

    pltpu.sync_copy(mmap_hbm, mm)
    pltpu.sync_copy(src_hbm.at[pl.ds(w * EPW, EPW)], sfl)
    pltpu.sync_copy(dst_hbm.at[pl.ds(w * EPW, EPW)], dfl)

    def _tile(t, cnt):
        idx = t * 16 + lane
        sv = plsc.load_gather(sfl, [idx])
        dv = plsc.load_gather(dfl, [idx])
        nsv = plsc.load_gather(mm, [sv])
        ndv = plsc.load_gather(mm, [dv])
        val = (nsv >= 0) & (ndv >= 0)
        pos = cnt + jnp.cumsum(jnp.where(val, 1, 0)) - 1
        plsc.store_scatter(nsb, [pos], nsv, mask=val)
        plsc.store_scatter(ndb, [pos], ndv, mask=val)
        return cnt + jnp.max(plsc.all_reduce_population_count(val))
    cnt = lax.fori_loop(0, EPW // 16, _tile, jnp.int32(0))

    for i in range(CH // 16):
        plsc.store_scatter(nsb, [cnt + i * 16 + lane], lane)
        plsc.store_scatter(ndb, [cnt + i * 16 + lane],
                           N + ((w * CH + i * 16 + lane) % NTRASH))

    nchunks = (cnt + (CH - 1)) // CH
    cbuf[...] = jnp.broadcast_to(nchunks, (16,)).astype(jnp.int32)
    pltpu.sync_copy(cbuf, cnt_out.at[w])
    pltpu.sync_copy(nsb.at[pl.ds(0, EPW)], ns_out.at[pl.ds(w * EPW, EPW)])
    pltpu.sync_copy(ndb.at[pl.ds(0, EPW)], nd_out.at[pl.ds(w * EPW, EPW)])


_BR = 1024
_G = NA // _BR


def _node_spec():
    return pl.BlockSpec((_BR, FEAT), lambda i: (i, 0))


def _col_spec():
    return pl.BlockSpec((_BR, 1), lambda i: (i, 0))


def _s_spec():
    return pl.BlockSpec((NC, _BR, FEAT), lambda i: (0, i, 0))


def _w_spec():
    return pl.BlockSpec((FEAT, FEAT), lambda i: (0, 0))


def _b_spec():
    return pl.BlockSpec((1, FEAT), lambda i: (0, 0))


def _tc_first_body(x_ref, w_ref, d_ref, h_ref, dinv_ref):
    deg = 1.0 + d_ref[0] + d_ref[1]
    dinv = 1.0 / jnp.sqrt(deg)
    h = jnp.dot(x_ref[...], w_ref[...], preferred_element_type=jnp.float32)
    h_ref[...] = h * dinv
    dinv_ref[...] = dinv


def _tc_first(x, w1, degp):
    return pl.pallas_call(
        _tc_first_body,
        grid=(_G,),
        in_specs=[_node_spec(), _w_spec(),
                  pl.BlockSpec((NC, _BR, 1), lambda i: (0, i, 0))],
        out_specs=[_node_spec(), _col_spec()],
        out_shape=[jax.ShapeDtypeStruct((NA, FEAT), jnp.float32),
                   jax.ShapeDtypeStruct((NA, 1), jnp.float32)],
    )(x, w1, degp)


def _tc_mid_body(h_ref, s_ref, dinv_ref, w_ref, b_ref, o_ref):
    dinv = dinv_ref[...]
    x = jnp.tanh(dinv * (h_ref[...] + s_ref[0] + s_ref[1]) + b_ref[...])
    o_ref[...] = jnp.dot(x, w_ref[...], preferred_element_type=jnp.float32) * dinv


def _tc_mid(h, s, dinv, w_next, b_prev):
    return pl.pallas_call(
        _tc_mid_body,
        grid=(_G,),
        in_specs=[_node_spec(), _s_spec(), _col_spec(), _w_spec(), _b_spec()],
        out_specs=_node_spec(),
        out_shape=jax.ShapeDtypeStruct((NA, FEAT), jnp.float32),
    )(h, s, dinv, w_next, b_prev)


def _tc_pool_a_body(h_ref, s_ref, dinv_ref, b_ref, pw_ref, x_ref, sc_ref):
    i = pl.program_id(0)
    x = jnp.tanh(dinv_ref[...] * (h_ref[...] + s_ref[0] + s_ref[1]) + b_ref[...])
    x_ref[...] = x
    pw = pw_ref[...]
    nrm = jnp.sqrt(jnp.sum(pw * pw))
    raw = jnp.sum(x * pw, axis=1, keepdims=True) / nrm
    rows = i * _BR + lax.broadcasted_iota(jnp.int32, (_BR, 1), 0)
    sc_ref[...] = jnp.where(rows < N, jnp.tanh(raw), -2.0)


def _tc_pool_a(h, s, dinv, b2, pw_row):
    return pl.pallas_call(
        _tc_pool_a_body,
        grid=(_G,),
        in_specs=[_node_spec(), _s_spec(), _col_spec(), _b_spec(),
                  pl.BlockSpec((1, FEAT), lambda i: (0, 0))],
        out_specs=[_node_spec(), _col_spec()],
        out_shape=[jax.ShapeDtypeStruct((NA, FEAT), jnp.float32),
                   jax.ShapeDtypeStruct((NA, 1), jnp.float32)],
    )(h, s, dinv, b2, pw_row)


def _tc_rank_body(sv_ref, su_ref, mmap_ref, gate_ref):
    i = pl.program_id(0)
    sv = sv_ref[...]
    vidx = i * _BR + lax.broadcasted_iota(jnp.int32, (_BR, 1), 0)
    svb = jnp.broadcast_to(sv, (_BR, 128))
    vidxb = jnp.broadcast_to(vidx, (_BR, 128))
    lanes = lax.broadcasted_iota(jnp.int32, (_BR, 128), 1)

    def _u(r, cnt):
        su = jnp.broadcast_to(su_ref[pl.ds(r, 1), :], (_BR, 128))
        uidx = r * 128 + lanes
        inc = (su > svb) | ((su == svb) & (uidx < vidxb))
        return cnt + jnp.where(inc, 1.0, 0.0)
    cnt = lax.fori_loop(0, NA // 128, _u, jnp.zeros((_BR, 128), jnp.float32))
    rank = jnp.sum(cnt, axis=1, keepdims=True)
    kept = rank < K
    mmap_ref[...] = jnp.where(kept, rank.astype(jnp.int32), -1)
    gate_ref[...] = jnp.where(kept, sv, 0.0)


def _tc_rank(score_col, score_row):
    return pl.pallas_call(
        _tc_rank_body,
        grid=(_G,),
        in_specs=[_col_spec(),
                  pl.BlockSpec((NA // 128, 128), lambda i: (0, 0))],
        out_specs=[_col_spec(), _col_spec()],
        out_shape=[jax.ShapeDtypeStruct((NA, 1), jnp.int32),
                   jax.ShapeDtypeStruct((NA, 1), jnp.float32)],
    )(score_col, score_row)


def _tc_pool_b_body(x_ref, g_ref, d_ref, w_ref, h_ref, dinv_ref):
    deg = 1.0 + d_ref[0] + d_ref[1]
    dinv = 1.0 / jnp.sqrt(deg)
    xo = x_ref[...] * g_ref[...]
    h_ref[...] = jnp.dot(xo, w_ref[...], preferred_element_type=jnp.float32) * dinv
    dinv_ref[...] = dinv


def _tc_pool_b(x2, gate, degp, w3):
    return pl.pallas_call(
        _tc_pool_b_body,
        grid=(_G,),
        in_specs=[_node_spec(), _col_spec(),
                  pl.BlockSpec((NC, _BR, 1), lambda i: (0, i, 0)), _w_spec()],
        out_specs=[_node_spec(), _col_spec()],
        out_shape=[jax.ShapeDtypeStruct((NA, FEAT), jnp.float32),
                   jax.ShapeDtypeStruct((NA, 1), jnp.float32)],
    )(x2, gate, degp, w3)


def _tc_final_body(h_ref, s_ref, dinv_ref, b_ref, o_ref):
    o_ref[...] = dinv_ref[...] * (h_ref[...] + s_ref[0] + s_ref[1]) + b_ref[...]


def _tc_final(h, s, dinv, b5):
    return pl.pallas_call(
        _tc_final_body,
        grid=(_G,),
        in_specs=[_node_spec(), _s_spec(), _col_spec(), _b_spec()],
        out_specs=_node_spec(),
        out_shape=jax.ShapeDtypeStruct((NA, FEAT), jnp.float32),
    )(h, s, dinv, b5)


def kernel(nodes, edges, batch, W1, b1, W2, b2, W3, b3, W4, b4, W5, b5, pw):
    del batch
    f32 = jnp.float32
    src = edges[0]
    dst = edges[1]

    npad = EP - E
    ar = jnp.arange(npad, dtype=jnp.int32)
    srcp = jnp.concatenate([src, ar % N]).reshape(EROWS, CH)
    dstp = jnp.concatenate([dst, N + (ar % NTRASH)]).reshape(EROWS, CH)
    full_cnt = jnp.full((NW, 16), RPW, dtype=jnp.int32)

    nodes_p = jnp.pad(nodes, ((0, NA - N), (0, 0)))
    b1r = b1.reshape(1, FEAT)
    b2r = b2.reshape(1, FEAT)
    b3r = b3.reshape(1, FEAT)
    b4r = b4.reshape(1, FEAT)
    b5r = b5.reshape(1, FEAT)
    pw_row = pw.reshape(1, FEAT)

    zrow = jnp.zeros((RPN, FEAT), dtype=f32)
    zdeg = jnp.zeros((RPN,), dtype=f32)

    deg0p = _sc_degree(dstp, full_cnt, zdeg)
    h1, dinv0 = _tc_first(nodes_p, W1, deg0p.reshape(NC, NA, 1))
    s1 = _sc_scatter(h1, srcp, dstp, full_cnt, zrow)
    h2 = _tc_mid(h1, s1, dinv0, W2, b1r)
    s2 = _sc_scatter(h2, srcp, dstp, full_cnt, zrow)

    x2, score = _tc_pool_a(h2, s2, dinv0, b2r, pw_row)
    mmap, gate = _tc_rank(score, score.reshape(NA // 128, 128))

    nsc, ndc, ccnt = _sc_compact(mmap.reshape(NA), srcp.reshape(EP), dstp.reshape(EP))
    nsc2 = nsc.reshape(EROWS, CH)
    ndc2 = ndc.reshape(EROWS, CH)
    deg1p = _sc_degree(ndc2, ccnt, zdeg)

    h3, dinv1 = _tc_pool_b(x2, gate, deg1p.reshape(NC, NA, 1), W3)
    s3 = _sc_scatter(h3, nsc2, ndc2, ccnt, zrow)
    h4 = _tc_mid(h3, s3, dinv1, W4, b3r)
    s4 = _sc_scatter(h4, nsc2, ndc2, ccnt, zrow)
    h5 = _tc_mid(h4, s4, dinv1, W5, b4r)
    s5 = _sc_scatter(h5, nsc2, ndc2, ccnt, zrow)
    out = _tc_final(h5, s5, dinv1, b5r)
    return out[:N].astype(f32)
